# baseline (speedup 1.0000x reference)
import functools

import jax
import jax.numpy as jnp
from jax import lax
from jax.experimental import pallas as pl
from jax.experimental.pallas import tpu as pltpu
from jax.experimental.pallas import tpu_sc as plsc

N_BINS = 15
N = 1048576
L = 16
NC, NS = 2, 16
NW = NC * NS
CHUNK = N // NW
BLK = 8192
NB = CHUNK // BLK
VECS = BLK // L
ACC = 768
QSTRIDE = 256

_mesh = plsc.VectorSubcoreMesh(
    core_axis_name="c", subcore_axis_name="s", num_cores=NC, num_subcores=NS
)
_params = pltpu.CompilerParams(needs_layout_passes=False)


@functools.partial(
    pl.kernel,
    out_type=jax.ShapeDtypeStruct((NW * ACC,), jnp.float32),
    mesh=_mesh,
    compiler_params=_params,
    scratch_types=[
        pltpu.VMEM((2, BLK), jnp.int32),
        pltpu.VMEM((2, BLK), jnp.float32),
        pltpu.VMEM((2, BLK), jnp.int32),
        pltpu.VMEM((L,), jnp.float32),
        pltpu.VMEM((ACC,), jnp.float32),
        pltpu.SemaphoreType.DMA,
        pltpu.SemaphoreType.DMA,
    ] + [pltpu.VMEM((QSTRIDE,), jnp.float32) for _ in range(12)],
)
def _hist_kernel(yhs_hbm, phs_hbm, ys_hbm, bounds_hbm, out_hbm,
                 yv, pv, vv, btab, acc, sem0, sem1, *banks):
    wid = lax.axis_index("c") * NS + lax.axis_index("s")
    pltpu.sync_copy(bounds_hbm, btab)

    zeros = jnp.zeros((L,), jnp.float32)
    for bank in banks:
        for v in range(QSTRIDE // L):
            bank[pl.ds(v * L, L)] = zeros

    lane = lax.iota(jnp.int32, L)
    ones = jnp.ones((L,), jnp.float32)
    sems = [sem0, sem1]
    NBANK = 4
    cnt_b = banks[0:4]
    cor_b = banks[4:8]
    cnf_b = banks[8:12]

    def start_block(b):
        base = wid * CHUNK + b * BLK
        s = b % 2
        sem = sems[s]
        return [
            pltpu.async_copy(yhs_hbm.at[pl.ds(base, BLK)], yv.at[s], sem),
            pltpu.async_copy(phs_hbm.at[pl.ds(base, BLK)], pv.at[s], sem),
            pltpu.async_copy(ys_hbm.at[pl.ds(base, BLK)], vv.at[s], sem),
        ]

    def do_vec(i, s, k):
        off = i * L
        p = pv[s, pl.ds(off, L)]
        yh = yv[s, pl.ds(off, L)]
        yy = vv[s, pl.ds(off, L)]
        t = p * jnp.float32(N_BINS)
        j0 = jnp.clip(t.astype(jnp.int32), 0, N_BINS - 1)
        lo = plsc.load_gather(btab, [j0])
        hi = plsc.load_gather(btab, [j0 + 1])
        j = j0 - (p <= lo).astype(jnp.int32) + (p > hi).astype(jnp.int32)
        j = jnp.clip(j, 0, N_BINS - 1)
        valid = p > jnp.float32(0.0)
        idx = j * L + lane
        correct = jnp.where(yh == yy, jnp.float32(1.0), jnp.float32(0.0))
        plsc.addupdate_scatter(cnt_b[k], [idx], ones, mask=valid)
        plsc.addupdate_scatter(cor_b[k], [idx], correct, mask=valid)
        plsc.addupdate_scatter(cnf_b[k], [idx], p, mask=valid)

    pending = start_block(0)
    for b in range(NB):
        for h in pending:
            h.wait()
        pending = start_block(b + 1) if b + 1 < NB else []
        s = b % 2

        def body(g, carry):
            for k in range(NBANK):
                do_vec(g * NBANK + k, s, k)
            return carry

        lax.fori_loop(0, VECS // NBANK, body, 0, unroll=2)

    for q in range(3):
        acc[pl.ds(q * QSTRIDE + N_BINS * L, L)] = zeros
    for q, qb in enumerate((cnt_b, cor_b, cnf_b)):
        for j in range(N_BINS):
            o = j * L
            vec = qb[0][pl.ds(o, L)] + qb[1][pl.ds(o, L)]
            vec = vec + qb[2][pl.ds(o, L)] + qb[3][pl.ds(o, L)]
            acc[pl.ds(q * QSTRIDE + o, L)] = vec

    pltpu.sync_copy(acc, out_hbm.at[pl.ds(wid * ACC, ACC)])


@functools.partial(
    pl.kernel,
    out_type=jax.ShapeDtypeStruct((L,), jnp.float32),
    mesh=_mesh,
    compiler_params=_params,
    scratch_types=[
        pltpu.VMEM((NW * ACC,), jnp.float32),
        pltpu.VMEM((ACC,), jnp.float32),
        pltpu.VMEM((L,), jnp.float32),
    ],
)
def _ece_kernel(parts_hbm, out_hbm, pv, acc, outv):
    wid = lax.axis_index("c") * NS + lax.axis_index("s")

    @pl.when(wid == 0)
    def _():
        pltpu.sync_copy(parts_hbm, pv)
        zeros = jnp.zeros((L,), jnp.float32)
        for v in range(ACC // L):
            acc[pl.ds(v * L, L)] = zeros

        def add_worker(w, carry):
            for v in range(ACC // L):
                o = v * L
                acc[pl.ds(o, L)] += pv[pl.ds(w * ACC + o, L)]
            return carry

        lax.fori_loop(0, NW, add_worker, 0)

        lane = lax.iota(jnp.int32, L)
        zeros = jnp.zeros((L,), jnp.float32)
        counts_v = zeros
        acc_v = zeros
        conf_v = zeros
        for j in range(N_BINS):
            sel = lane == j
            c = jnp.sum(acc[pl.ds(j * L, L)])
            a = jnp.sum(acc[pl.ds(QSTRIDE + j * L, L)])
            f = jnp.sum(acc[pl.ds(2 * QSTRIDE + j * L, L)])
            counts_v = jnp.where(sel, jnp.broadcast_to(c, (L,)), counts_v)
            acc_v = jnp.where(sel, jnp.broadcast_to(a, (L,)), acc_v)
            conf_v = jnp.where(sel, jnp.broadcast_to(f, (L,)), conf_v)
        ones = jnp.ones((L,), jnp.float32)
        ind = counts_v > jnp.float32(0.0)
        safe = jnp.where(ind, counts_v, ones)
        mean_acc = jnp.where(ind, acc_v / safe, acc_v)
        mean_conf = jnp.where(ind, conf_v / safe, conf_v)
        num = jnp.sum(counts_v * jnp.abs(mean_acc - mean_conf))
        tot = jnp.sum(counts_v)
        outv[...] = jnp.broadcast_to(num, (L,)) / jnp.broadcast_to(tot, (L,))
        pltpu.sync_copy(outv, out_hbm)


@jax.jit
def kernel(yhs, phs, ys):
    bounds = jnp.linspace(0.0, 1.0, N_BINS + 1).astype(jnp.float32)
    parts = _hist_kernel(yhs, phs, ys, bounds)
    ece_vec = _ece_kernel(parts)
    return ece_vec[0]


# speedup vs baseline: 2.1871x; 2.1871x over previous
import functools

import jax
import jax.numpy as jnp
from jax import lax
from jax.experimental import pallas as pl
from jax.experimental.pallas import tpu as pltpu
from jax.experimental.pallas import tpu_sc as plsc

N_BINS = 15
N = 1048576
L = 16
NC, NS = 2, 16
NW = NC * NS
CHUNK = N // NW
BLK = 8192
NB = CHUNK // BLK
VECS = BLK // L
PROW = 48

_mesh = plsc.VectorSubcoreMesh(
    core_axis_name="c", subcore_axis_name="s", num_cores=NC, num_subcores=NS
)
_params = pltpu.CompilerParams(needs_layout_passes=False)


@functools.partial(
    pl.kernel,
    out_type=jax.ShapeDtypeStruct((NW * PROW,), jnp.float32),
    mesh=_mesh,
    compiler_params=_params,
    scratch_types=[
        pltpu.VMEM((2, BLK), jnp.int32),
        pltpu.VMEM((2, BLK), jnp.float32),
        pltpu.VMEM((2, BLK), jnp.int32),
        pltpu.VMEM((L,), jnp.float32),
        pltpu.VMEM((N_BINS * L,), jnp.float32),
        pltpu.VMEM((N_BINS * L,), jnp.float32),
        pltpu.VMEM((N_BINS * L,), jnp.float32),
        pltpu.VMEM((PROW,), jnp.float32),
        pltpu.SemaphoreType.DMA,
        pltpu.SemaphoreType.DMA,
    ],
)
def _hist_kernel(yhs_hbm, phs_hbm, ys_hbm, bounds_hbm, out_hbm,
                 yv, pv, vv, btab, cnt, cor, cnf, row, sem0, sem1):
    wid = lax.axis_index("c") * NS + lax.axis_index("s")
    pltpu.sync_copy(bounds_hbm, btab)
    bvec = btab[...]

    zeros = jnp.zeros((L,), jnp.float32)
    for v in range(N_BINS):
        cnt[pl.ds(v * L, L)] = zeros
        cor[pl.ds(v * L, L)] = zeros
        cnf[pl.ds(v * L, L)] = zeros

    lane = lax.iota(jnp.int32, L)
    ones = jnp.ones((L,), jnp.float32)
    sems = [sem0, sem1]

    def start_block(b):
        base = wid * CHUNK + b * BLK
        s = b % 2
        sem = sems[s]
        return [
            pltpu.async_copy(yhs_hbm.at[pl.ds(base, BLK)], yv.at[s], sem),
            pltpu.async_copy(phs_hbm.at[pl.ds(base, BLK)], pv.at[s], sem),
            pltpu.async_copy(ys_hbm.at[pl.ds(base, BLK)], vv.at[s], sem),
        ]

    pending = start_block(0)
    for b in range(NB):
        for h in pending:
            h.wait()
        pending = start_block(b + 1) if b + 1 < NB else []
        s = b % 2

        @plsc.parallel_loop(0, VECS, unroll=4)
        def _(i):
            off = i * L
            p = pv[s, pl.ds(off, L)]
            yh = yv[s, pl.ds(off, L)]
            yy = vv[s, pl.ds(off, L)]
            t = p * jnp.float32(N_BINS)
            j0 = t.astype(jnp.int32)
            lo = jnp.take_along_axis(bvec, j0, axis=0)
            hi = jnp.take_along_axis(bvec, jnp.bitwise_and(j0 + 1, 15), axis=0)
            j = j0 - (p <= lo).astype(jnp.int32) + (p > hi).astype(jnp.int32)
            j = jnp.clip(j, 0, N_BINS - 1)
            valid = p > jnp.float32(0.0)
            idx = j * L + lane
            correct = jnp.where(yh == yy, jnp.float32(1.0), jnp.float32(0.0))
            plsc.addupdate_scatter(cnt, [idx], ones, mask=valid)
            plsc.addupdate_scatter(cor, [idx], correct, mask=valid)
            plsc.addupdate_scatter(cnf, [idx], p, mask=valid)

    for q, src in enumerate((cnt, cor, cnf)):
        vec = zeros
        for j in range(N_BINS):
            sj = jnp.sum(src[pl.ds(j * L, L)])
            vec = jnp.where(lane == j, jnp.broadcast_to(sj, (L,)), vec)
        row[pl.ds(q * L, L)] = vec

    pltpu.sync_copy(row, out_hbm.at[pl.ds(wid * PROW, PROW)])


@functools.partial(
    pl.kernel,
    out_type=jax.ShapeDtypeStruct((L,), jnp.float32),
    mesh=_mesh,
    compiler_params=_params,
    scratch_types=[
        pltpu.VMEM((NW * PROW,), jnp.float32),
        pltpu.VMEM((L,), jnp.float32),
    ],
)
def _ece_kernel(parts_hbm, out_hbm, pv, outv):
    wid = lax.axis_index("c") * NS + lax.axis_index("s")

    @pl.when(wid == 0)
    def _():
        pltpu.sync_copy(parts_hbm, pv)
        zeros = jnp.zeros((L,), jnp.float32)

        def add_worker(w, carry):
            c, a, f = carry
            o = w * PROW
            c = c + pv[pl.ds(o, L)]
            a = a + pv[pl.ds(o + L, L)]
            f = f + pv[pl.ds(o + 2 * L, L)]
            return (c, a, f)

        counts_v, acc_v, conf_v = lax.fori_loop(
            0, NW, add_worker, (zeros, zeros, zeros), unroll=4
        )

        ones = jnp.ones((L,), jnp.float32)
        ind = counts_v > jnp.float32(0.0)
        safe = jnp.where(ind, counts_v, ones)
        mean_acc = jnp.where(ind, acc_v / safe, acc_v)
        mean_conf = jnp.where(ind, conf_v / safe, conf_v)
        num = jnp.sum(counts_v * jnp.abs(mean_acc - mean_conf))
        tot = jnp.sum(counts_v)
        outv[...] = jnp.broadcast_to(num, (L,)) / jnp.broadcast_to(tot, (L,))
        pltpu.sync_copy(outv, out_hbm)


@jax.jit
def kernel(yhs, phs, ys):
    bounds = jnp.linspace(0.0, 1.0, N_BINS + 1).astype(jnp.float32)
    parts = _hist_kernel(yhs, phs, ys, bounds)
    ece_vec = _ece_kernel(parts)
    return ece_vec[0]
